# baseline (speedup 1.0000x reference)
import jax
import jax.numpy as jnp
from jax import lax
from jax.experimental import pallas as pl
from jax.experimental.pallas import tpu as pltpu
from jax.experimental.pallas import tpu_sc as plsc

N = 10000
E = 320000
D = 128

NC = 2
NS = 16
NW = NC * NS
C = 128
R = E // C
Q, REM = divmod(R, NW)
RW = Q + 1
NR = 80

_mesh = plsc.VectorSubcoreMesh(core_axis_name="c", subcore_axis_name="s")
_params = pltpu.CompilerParams(needs_layout_passes=False)


def _worker_slice(wid):
    start = wid * Q + jnp.minimum(wid, REM)
    nrows = Q + jnp.where(wid < REM, 1, 0)
    return start, nrows


def _seg_sum_body(x_hbm, src_hbm, dst_hbm, agg_out,
                  src_i, dst2, rows2, agg_sh, isem, gsem, dsem, ssem):
    cid = lax.axis_index("c")
    sid = lax.axis_index("s")
    wid = sid * NC + cid

    z16 = jnp.zeros((16,), jnp.float32)

    def zrow(i, _):
        for j in range(D // 16):
            rows2[0, i, pl.ds(16 * j, 16)] = z16
        return 0

    lax.fori_loop(0, C, zrow, 0)

    gq, grem = divmod(N // 8, NS)
    gstart = sid * gq + jnp.minimum(sid, grem)
    ng = gq + jnp.where(sid < grem, 1, 0)

    def zshared(k, _):
        base = (gstart + k) * 8
        pltpu.sync_copy(rows2.at[0, pl.ds(0, 8)], agg_sh.at[pl.ds(base, 8)])
        return 0

    lax.fori_loop(0, ng, zshared, 0)
    plsc.subcore_barrier()

    start, nrows = _worker_slice(wid)
    start_off = start * C

    def fire_src(chunk):
        ph = lax.rem(chunk, 3)
        pltpu.async_copy(src_hbm.at[pl.ds(start_off + chunk * C, C)],
                         src_i.at[ph], isem.at[ph])

    def fire_dst(chunk):
        ph = lax.rem(chunk, 3)
        pltpu.async_copy(dst_hbm.at[pl.ds(start_off + chunk * C, C)],
                         dst2.at[ph], dsem.at[ph])

    def drain_src(chunk):
        ph = lax.rem(chunk, 3)
        pltpu.make_async_copy(src_hbm.at[pl.ds(start_off + chunk * C, C)],
                              src_i.at[ph], isem.at[ph]).wait()

    def fire_gather(chunk):
        ph = lax.rem(chunk, 3)
        pltpu.async_copy(x_hbm.at[src_i.at[ph]], rows2.at[ph], gsem.at[ph])

    def drain_scatter(chunk):
        ph = lax.rem(chunk, 3)
        pltpu.make_async_copy(rows2.at[ph], agg_sh.at[dst2.at[ph]],
                              ssem.at[ph]).wait()

    fire_src(jnp.int32(0))
    fire_src(jnp.int32(1))
    fire_dst(jnp.int32(0))
    drain_src(jnp.int32(0))
    fire_gather(jnp.int32(0))

    def body(i, _):
        ph = lax.rem(i, 3)

        @pl.when(i >= 2)
        def _drain_old_scatter():
            drain_scatter(i - 2)

        @pl.when(i + 2 < nrows)
        def _prefetch_src():
            fire_src(i + 2)

        @pl.when(i + 1 < nrows)
        def _prefetch():
            drain_src(i + 1)
            fire_gather(i + 1)
            fire_dst(i + 1)

        pltpu.make_async_copy(x_hbm.at[src_i.at[ph]], rows2.at[ph],
                              gsem.at[ph]).wait()
        pltpu.make_async_copy(dst_hbm.at[pl.ds(start_off + i * C, C)],
                              dst2.at[ph], dsem.at[ph]).wait()
        pltpu.async_copy(rows2.at[ph], agg_sh.at[dst2.at[ph]], ssem.at[ph],
                         add=True)
        return 0

    lax.fori_loop(0, nrows, body, 0)
    drain_scatter(nrows - 2)
    drain_scatter(nrows - 1)
    plsc.subcore_barrier()

    def wout(k, _):
        base = (gstart + k) * 8
        pltpu.sync_copy(agg_sh.at[pl.ds(base, 8)],
                        agg_out.at[cid, pl.ds(base, 8)])
        return 0

    lax.fori_loop(0, ng, wout, 0)


_seg_sum = pl.kernel(
    _seg_sum_body,
    out_type=jax.ShapeDtypeStruct((NC, N, D), jnp.float32),
    mesh=_mesh,
    scratch_types=[
        pltpu.VMEM((3, C), jnp.int32),
        pltpu.VMEM((3, C), jnp.int32),
        pltpu.VMEM((3, C, D), jnp.float32),
        pltpu.VMEM_SHARED((N, D), jnp.float32),
        pltpu.SemaphoreType.DMA((3,)),
        pltpu.SemaphoreType.DMA((3,)),
        pltpu.SemaphoreType.DMA((3,)),
        pltpu.SemaphoreType.DMA((3,)),
    ],
    compiler_params=_params,
    name="sc_segment_sum",
)


def _deg_body(dst_hbm, deg_out, dst_i, deg_loc, rowidx_v, deg_sh, sem):
    cid = lax.axis_index("c")
    sid = lax.axis_index("s")
    wid = sid * NC + cid

    z16 = jnp.zeros((16,), jnp.float32)

    def zdeg(i, _):
        for j in range(D // 16):
            deg_loc[i, pl.ds(16 * j, 16)] = z16
        return 0

    lax.fori_loop(0, NR, zdeg, 0)

    def fillrow(k, _):
        rowidx_v[pl.ds(16 * k, 16)] = lax.iota(jnp.int32, 16) + 16 * k
        return 0

    lax.fori_loop(0, NR // 16, fillrow, 0)

    @pl.when(sid == 0)
    def _zdegsh():
        pltpu.sync_copy(deg_loc.at[pl.ds(0, NR)], deg_sh.at[pl.ds(0, NR)])

    plsc.subcore_barrier()

    start, nrows = _worker_slice(wid)
    start_off = start * C
    pltpu.sync_copy(dst_hbm.at[pl.ds(start_off, Q * C)],
                    dst_i.at[pl.ds(0, Q * C)])

    @pl.when(nrows == Q + 1)
    def _last_chunk_idx():
        pltpu.sync_copy(dst_hbm.at[pl.ds(start_off + Q * C, C)],
                        dst_i.at[pl.ds(Q * C, C)])

    o16 = jnp.ones((16,), jnp.float32)

    def body(i, _):
        for j in range(C // 16):
            d16 = dst_i[pl.ds(i * C + 16 * j, 16)]
            r16 = lax.shift_right_logical(d16, 7)
            c16 = lax.bitwise_and(d16, 127)
            plsc.addupdate_scatter(deg_loc, [r16, c16], o16)
        return 0

    lax.fori_loop(0, nrows, body, 0)
    pltpu.sync_copy(deg_loc, deg_sh.at[rowidx_v], add=True)
    plsc.subcore_barrier()

    @pl.when(sid < NR // 8)
    def _wdeg():
        pltpu.sync_copy(deg_sh.at[pl.ds(sid * 8, 8)],
                        deg_out.at[cid, pl.ds(sid * 8, 8)])


_deg_kernel = pl.kernel(
    _deg_body,
    out_type=jax.ShapeDtypeStruct((NC, NR, D), jnp.float32),
    mesh=_mesh,
    scratch_types=[
        pltpu.VMEM((RW * C,), jnp.int32),
        pltpu.VMEM((NR, D), jnp.float32),
        pltpu.VMEM((NR,), jnp.int32),
        pltpu.VMEM_SHARED((NR, D), jnp.float32),
        pltpu.SemaphoreType.DMA,
    ],
    compiler_params=_params,
    name="sc_degrees",
)

_BN_S = 1.0 / (1.0 + 1e-5) ** 0.5


def _tcr_body(h, W, hr_out):
    hr_out[...] = lax.dot_general(h[...], W[...], (((1,), (1,)), ((), ())),
                                  preferred_element_type=jnp.float32)


def _tc1_body(aggp, deg, xr, W1l, b1, g1, be1, h1_out):
    agg = aggp[0] + aggp[1]
    mean = agg * (1.0 / jnp.maximum(deg[...], 1.0))
    t = lax.dot_general(mean, W1l[...], (((1,), (1,)), ((), ())),
                        preferred_element_type=jnp.float32) + xr[...]
    t = (t + b1[...]) * (g1[...] * _BN_S) + be1[...]
    h1_out[...] = jnp.maximum(t, 0.0)


def _tc2_body(aggp, deg, h1, h1r, W2l, b2, g2, be2, Wo, bo, out):
    agg = aggp[0] + aggp[1]
    mean = agg * (1.0 / jnp.maximum(deg[...], 1.0))
    t = lax.dot_general(mean, W2l[...], (((1,), (1,)), ((), ())),
                        preferred_element_type=jnp.float32) + h1r[...]
    t = (t + b2[...]) * (g2[...] * _BN_S) + be2[...]
    h = h1[...] + jnp.maximum(t, 0.0)
    out[...] = lax.dot_general(h, Wo[...], (((1,), (1,)), ((), ())),
                               preferred_element_type=jnp.float32) + bo[...]


_BLK = 2000
_GRID = N // _BLK


def _row_spec(width=D):
    return pl.BlockSpec((_BLK, width), lambda i: (i, 0))


def _part_spec(width):
    return pl.BlockSpec((NC, _BLK, width), lambda i: (0, i, 0))


def _full_spec(shape):
    return pl.BlockSpec(shape, lambda i: tuple(0 for _ in shape))


_tcr = pl.pallas_call(
    _tcr_body,
    grid=(_GRID,),
    in_specs=[_row_spec(), _full_spec((D, D))],
    out_specs=_row_spec(),
    out_shape=jax.ShapeDtypeStruct((N, D), jnp.float32),
)

_tc1 = pl.pallas_call(
    _tc1_body,
    grid=(_GRID,),
    in_specs=[
        _part_spec(D), _row_spec(), _row_spec(),
        _full_spec((D, D)),
        _full_spec((1, D)), _full_spec((1, D)), _full_spec((1, D)),
    ],
    out_specs=_row_spec(),
    out_shape=jax.ShapeDtypeStruct((N, D), jnp.float32),
)

_tc2 = pl.pallas_call(
    _tc2_body,
    grid=(_GRID,),
    in_specs=[
        _part_spec(D), _row_spec(), _row_spec(), _row_spec(),
        _full_spec((D, D)),
        _full_spec((1, D)), _full_spec((1, D)), _full_spec((1, D)),
        _full_spec((D, D)), _full_spec((1, D)),
    ],
    out_specs=_row_spec(),
    out_shape=jax.ShapeDtypeStruct((N, D), jnp.float32),
)


@jax.jit
def kernel(x, edge_index, W1l, W1r, b1, g1, be1, W2l, W2r, b2, g2, be2, Wo, bo):
    src1 = edge_index[0]
    dst1 = edge_index[1]
    degp = _deg_kernel(dst1)
    aggp1 = _seg_sum(x, src1, dst1)
    deg = jnp.broadcast_to(
        (degp[0] + degp[1]).reshape(NR * D)[:N, None], (N, D))
    xr = _tcr(x, W1r)
    h1 = _tc1(aggp1, deg, xr, W1l,
              b1.reshape(1, D), g1.reshape(1, D), be1.reshape(1, D))
    aggp2 = _seg_sum(h1, src1, dst1)
    h1r = _tcr(h1, W2r)
    return _tc2(aggp2, deg, h1, h1r, W2l,
                b2.reshape(1, D), g2.reshape(1, D), be2.reshape(1, D),
                Wo, bo.reshape(1, D))


# speedup vs baseline: 1.0045x; 1.0045x over previous
import jax
import jax.numpy as jnp
from jax import lax
from jax.experimental import pallas as pl
from jax.experimental.pallas import tpu as pltpu
from jax.experimental.pallas import tpu_sc as plsc

N = 10000
E = 320000
D = 128

NC = 2
NS = 16
NW = NC * NS
C = 128
R = E // C
Q, REM = divmod(R, NW)
RW = Q + 1
NR = 80

_mesh = plsc.VectorSubcoreMesh(core_axis_name="c", subcore_axis_name="s")
_params = pltpu.CompilerParams(needs_layout_passes=False)


def _worker_slice(wid):
    start = wid * Q + jnp.minimum(wid, REM)
    nrows = Q + jnp.where(wid < REM, 1, 0)
    return start, nrows


def _seg_sum_body(x_hbm, src_hbm, dst_hbm, agg_out,
                  src_i, dst2, rows2, agg_sh, isem, gsem, dsem, ssem):
    cid = lax.axis_index("c")
    sid = lax.axis_index("s")
    wid = sid * NC + cid

    z16 = jnp.zeros((16,), jnp.float32)

    def zrow(i, _):
        for j in range(D // 16):
            rows2[0, i, pl.ds(16 * j, 16)] = z16
        return 0

    lax.fori_loop(0, C, zrow, 0)

    gq, grem = divmod(N // 8, NS)
    gstart = sid * gq + jnp.minimum(sid, grem)
    ng = gq + jnp.where(sid < grem, 1, 0)

    def zshared(k, _):
        base = (gstart + k) * 8
        pltpu.sync_copy(rows2.at[0, pl.ds(0, 8)], agg_sh.at[pl.ds(base, 8)])
        return 0

    lax.fori_loop(0, ng, zshared, 0)
    plsc.subcore_barrier()

    start, nrows = _worker_slice(wid)
    start_off = start * C

    def fire_src(chunk):
        ph = lax.rem(chunk, 3)
        pltpu.async_copy(src_hbm.at[pl.ds(start_off + chunk * C, C)],
                         src_i.at[ph], isem.at[ph])

    def fire_dst(chunk):
        ph = lax.rem(chunk, 3)
        pltpu.async_copy(dst_hbm.at[pl.ds(start_off + chunk * C, C)],
                         dst2.at[ph], dsem.at[ph])

    def drain_src(chunk):
        ph = lax.rem(chunk, 3)
        pltpu.make_async_copy(src_hbm.at[pl.ds(start_off + chunk * C, C)],
                              src_i.at[ph], isem.at[ph]).wait()

    def fire_gather(chunk):
        ph = lax.rem(chunk, 3)
        pltpu.async_copy(x_hbm.at[src_i.at[ph]], rows2.at[ph], gsem.at[ph])

    def drain_scatter(chunk):
        ph = lax.rem(chunk, 3)
        pltpu.make_async_copy(rows2.at[ph], agg_sh.at[dst2.at[ph]],
                              ssem.at[ph]).wait()

    fire_src(jnp.int32(0))
    fire_src(jnp.int32(1))
    fire_dst(jnp.int32(0))
    drain_src(jnp.int32(0))
    fire_gather(jnp.int32(0))

    def body(i, _):
        ph = lax.rem(i, 3)

        @pl.when(i >= 2)
        def _drain_old_scatter():
            drain_scatter(i - 2)

        @pl.when(i + 2 < nrows)
        def _prefetch_src():
            fire_src(i + 2)

        @pl.when(i + 1 < nrows)
        def _prefetch():
            drain_src(i + 1)
            fire_gather(i + 1)
            fire_dst(i + 1)

        pltpu.make_async_copy(x_hbm.at[src_i.at[ph]], rows2.at[ph],
                              gsem.at[ph]).wait()
        pltpu.make_async_copy(dst_hbm.at[pl.ds(start_off + i * C, C)],
                              dst2.at[ph], dsem.at[ph]).wait()
        pltpu.async_copy(rows2.at[ph], agg_sh.at[dst2.at[ph]], ssem.at[ph],
                         add=True)
        return 0

    lax.fori_loop(0, nrows, body, 0)
    drain_scatter(nrows - 2)
    drain_scatter(nrows - 1)
    plsc.subcore_barrier()

    def wout(k, _):
        base = (gstart + k) * 8
        pltpu.sync_copy(agg_sh.at[pl.ds(base, 8)],
                        agg_out.at[cid, pl.ds(base, 8)])
        return 0

    lax.fori_loop(0, ng, wout, 0)


_seg_sum = pl.kernel(
    _seg_sum_body,
    out_type=jax.ShapeDtypeStruct((NC, N, D), jnp.float32),
    mesh=_mesh,
    scratch_types=[
        pltpu.VMEM((3, C), jnp.int32),
        pltpu.VMEM((3, C), jnp.int32),
        pltpu.VMEM((3, C, D), jnp.float32),
        pltpu.VMEM_SHARED((N, D), jnp.float32),
        pltpu.SemaphoreType.DMA((3,)),
        pltpu.SemaphoreType.DMA((3,)),
        pltpu.SemaphoreType.DMA((3,)),
        pltpu.SemaphoreType.DMA((3,)),
    ],
    compiler_params=_params,
    name="sc_segment_sum",
)


def _deg_body(dst_hbm, deg_out, dst_i, deg_loc, rowidx_v, deg_sh, sem):
    cid = lax.axis_index("c")
    sid = lax.axis_index("s")
    wid = sid * NC + cid

    z16 = jnp.zeros((16,), jnp.float32)

    def zdeg(i, _):
        for j in range(D // 16):
            deg_loc[i, pl.ds(16 * j, 16)] = z16
        return 0

    lax.fori_loop(0, NR, zdeg, 0)

    def fillrow(k, _):
        rowidx_v[pl.ds(16 * k, 16)] = lax.iota(jnp.int32, 16) + 16 * k
        return 0

    lax.fori_loop(0, NR // 16, fillrow, 0)

    @pl.when(sid == 0)
    def _zdegsh():
        pltpu.sync_copy(deg_loc.at[pl.ds(0, NR)], deg_sh.at[pl.ds(0, NR)])

    plsc.subcore_barrier()

    start, nrows = _worker_slice(wid)
    start_off = start * C
    pltpu.sync_copy(dst_hbm.at[pl.ds(start_off, Q * C)],
                    dst_i.at[pl.ds(0, Q * C)])

    @pl.when(nrows == Q + 1)
    def _last_chunk_idx():
        pltpu.sync_copy(dst_hbm.at[pl.ds(start_off + Q * C, C)],
                        dst_i.at[pl.ds(Q * C, C)])

    o16 = jnp.ones((16,), jnp.float32)

    def body(i, _):
        for j in range(C // 16):
            d16 = dst_i[pl.ds(i * C + 16 * j, 16)]
            r16 = lax.shift_right_logical(d16, 7)
            c16 = lax.bitwise_and(d16, 127)
            plsc.addupdate_scatter(deg_loc, [r16, c16], o16)
        return 0

    lax.fori_loop(0, nrows, body, 0)
    pltpu.sync_copy(deg_loc, deg_sh.at[rowidx_v], add=True)
    plsc.subcore_barrier()

    @pl.when(sid < NR // 8)
    def _wdeg():
        pltpu.sync_copy(deg_sh.at[pl.ds(sid * 8, 8)],
                        deg_out.at[cid, pl.ds(sid * 8, 8)])


_deg_kernel = pl.kernel(
    _deg_body,
    out_type=jax.ShapeDtypeStruct((NC, NR, D), jnp.float32),
    mesh=_mesh,
    scratch_types=[
        pltpu.VMEM((RW * C,), jnp.int32),
        pltpu.VMEM((NR, D), jnp.float32),
        pltpu.VMEM((NR,), jnp.int32),
        pltpu.VMEM_SHARED((NR, D), jnp.float32),
        pltpu.SemaphoreType.DMA,
    ],
    compiler_params=_params,
    name="sc_degrees",
)

_BN_S = 1.0 / (1.0 + 1e-5) ** 0.5


def _tc1_body(aggp, deg, x, W1l, W1r, b1, g1, be1, h1_out):
    agg = aggp[0] + aggp[1]
    mean = agg * (1.0 / jnp.maximum(deg[...], 1.0))
    t = lax.dot_general(mean, W1l[...], (((1,), (1,)), ((), ())),
                        preferred_element_type=jnp.float32)
    t = t + lax.dot_general(x[...], W1r[...], (((1,), (1,)), ((), ())),
                            preferred_element_type=jnp.float32)
    t = (t + b1[...]) * (g1[...] * _BN_S) + be1[...]
    h1_out[...] = jnp.maximum(t, 0.0)


def _tc2_body(aggp, deg, h1, W2l, W2r, b2, g2, be2, Wo, bo, out):
    agg = aggp[0] + aggp[1]
    mean = agg * (1.0 / jnp.maximum(deg[...], 1.0))
    t = lax.dot_general(mean, W2l[...], (((1,), (1,)), ((), ())),
                        preferred_element_type=jnp.float32)
    t = t + lax.dot_general(h1[...], W2r[...], (((1,), (1,)), ((), ())),
                            preferred_element_type=jnp.float32)
    t = (t + b2[...]) * (g2[...] * _BN_S) + be2[...]
    h = h1[...] + jnp.maximum(t, 0.0)
    out[...] = lax.dot_general(h, Wo[...], (((1,), (1,)), ((), ())),
                               preferred_element_type=jnp.float32) + bo[...]


_BLK = 2000
_GRID = N // _BLK


def _row_spec(width=D):
    return pl.BlockSpec((_BLK, width), lambda i: (i, 0))


def _part_spec(width):
    return pl.BlockSpec((NC, _BLK, width), lambda i: (0, i, 0))


def _full_spec(shape):
    return pl.BlockSpec(shape, lambda i: tuple(0 for _ in shape))


_tc1 = pl.pallas_call(
    _tc1_body,
    grid=(_GRID,),
    in_specs=[
        _part_spec(D), _row_spec(), _row_spec(),
        _full_spec((D, D)), _full_spec((D, D)),
        _full_spec((1, D)), _full_spec((1, D)), _full_spec((1, D)),
    ],
    out_specs=_row_spec(),
    out_shape=jax.ShapeDtypeStruct((N, D), jnp.float32),
)

_tc2 = pl.pallas_call(
    _tc2_body,
    grid=(_GRID,),
    in_specs=[
        _part_spec(D), _row_spec(), _row_spec(),
        _full_spec((D, D)), _full_spec((D, D)),
        _full_spec((1, D)), _full_spec((1, D)), _full_spec((1, D)),
        _full_spec((D, D)), _full_spec((1, D)),
    ],
    out_specs=_row_spec(),
    out_shape=jax.ShapeDtypeStruct((N, D), jnp.float32),
)


@jax.jit
def kernel(x, edge_index, W1l, W1r, b1, g1, be1, W2l, W2r, b2, g2, be2, Wo, bo):
    src1 = edge_index[0]
    dst1 = edge_index[1]
    degp = _deg_kernel(dst1)
    aggp1 = _seg_sum(x, src1, dst1)
    deg = jnp.broadcast_to(
        (degp[0] + degp[1]).reshape(NR * D)[:N, None], (N, D))
    h1 = _tc1(aggp1, deg, x, W1l, W1r,
              b1.reshape(1, D), g1.reshape(1, D), be1.reshape(1, D))
    aggp2 = _seg_sum(h1, src1, dst1)
    return _tc2(aggp2, deg, h1, W2l, W2r,
                b2.reshape(1, D), g2.reshape(1, D), be2.reshape(1, D),
                Wo, bo.reshape(1, D))
